# Initial kernel scaffold; baseline (speedup 1.0000x reference)
import functools

import jax
import jax.numpy as jnp
from jax import lax
from jax.experimental import pallas as pl
from jax.experimental.pallas import tpu as pltpu
from jax.experimental.pallas import tpu_sc as plsc

NUM_CONS = 100000
NUM_RELS = 1000
DIM = 128
B = 4096
NEG = 10
REG_SCALE = 0.0005
P_NEG = 1.0

NC = 2
NS = 16
L = 16
NW = NC * NS
CH = 128
POS_PER_W = B // NW
NEG_PER_W = (B * NEG) // NW
NEG_CHUNKS = NEG_PER_W // CH
DG = DIM // L


def _sc_body(h_hbm, r_hbm, t_hbm, w_hbm,
             hn_e_hbm, hn_r_hbm, hn_t_hbm,
             tn_e_hbm, tn_r_hbm, tn_t_hbm,
             ent_re_hbm, ent_im_hbm, rel_re_hbm, rel_im_hbm,
             lwb_hbm,
             out_hbm,
             ia_v, ib_v, ic_v, w_v,
             hre_v, him_v, rre_v, rim_v, tre_v, tim_v,
             sc_v, lwb_v, part_v, sem):
    cid = lax.axis_index("c")
    sid = lax.axis_index("s")
    wid = sid * NC + cid

    pltpu.sync_copy(lwb_hbm, lwb_v)
    lw = lwb_v[0, :]
    lb = lwb_v[1, :]

    def gather_chunk(eidx_hbm, ridx_hbm, tidx_hbm, base):
        base = pl.multiple_of(base, 8)
        pltpu.sync_copy(eidx_hbm.at[pl.ds(base, CH)], ia_v)
        pltpu.sync_copy(ridx_hbm.at[pl.ds(base, CH)], ib_v)
        pltpu.sync_copy(tidx_hbm.at[pl.ds(base, CH)], ic_v)
        cps = [
            pltpu.make_async_copy(ent_re_hbm.at[ia_v], hre_v, sem),
            pltpu.make_async_copy(ent_im_hbm.at[ia_v], him_v, sem),
            pltpu.make_async_copy(rel_re_hbm.at[ib_v], rre_v, sem),
            pltpu.make_async_copy(rel_im_hbm.at[ib_v], rim_v, sem),
            pltpu.make_async_copy(ent_re_hbm.at[ic_v], tre_v, sem),
            pltpu.make_async_copy(ent_im_hbm.at[ic_v], tim_v, sem),
        ]
        for cp in cps:
            cp.start()
        for cp in cps:
            cp.wait()

    def score_rows(with_reg, reg0):
        def row(i, reg):
            acc = jnp.zeros((L,), jnp.float32)
            for d in range(DG):
                sl = pl.ds(d * L, L)
                a = hre_v[i, sl]
                bb = him_v[i, sl]
                c = rre_v[i, sl]
                e = rim_v[i, sl]
                f = tre_v[i, sl]
                g = tim_v[i, sl]
                acc = acc + c * (a * f + bb * g) + e * (a * g - bb * f)
                if with_reg:
                    reg = reg + (a * a + bb * bb + c * c
                                 + e * e + f * f + g * g)
            sc_v[i] = jnp.sum(acc)
            return reg
        return lax.fori_loop(0, CH, row, reg0)

    def loss_pass(positive, acc0):
        acc = acc0
        for g in range(CH // L):
            sl = pl.ds(g * L, L)
            z = sc_v[sl] * lw + lb
            p = 1.0 / (1.0 + jnp.exp(-z))
            if positive:
                dd = p - w_v[sl]
            else:
                dd = p
            acc = acc + dd * dd
        return acc

    zero = jnp.zeros((L,), jnp.float32)

    pos_base = pl.multiple_of(wid * POS_PER_W, 8)
    pltpu.sync_copy(w_hbm.at[pl.ds(pos_base, CH)], w_v)
    gather_chunk(h_hbm, r_hbm, t_hbm, pos_base)
    reg_acc = score_rows(True, zero)
    pos_acc = loss_pass(True, zero)

    def neg_loop(eidx, ridx, tidx):
        def chunk(c, acc):
            gather_chunk(eidx, ridx, tidx, wid * NEG_PER_W + c * CH)
            score_rows(False, zero)
            return loss_pass(False, acc)
        return lax.fori_loop(0, NEG_CHUNKS, chunk, zero)

    hn_acc = neg_loop(hn_e_hbm, hn_r_hbm, hn_t_hbm)
    tn_acc = neg_loop(tn_e_hbm, tn_r_hbm, tn_t_hbm)

    partial = (pos_acc
               + (hn_acc + tn_acc) * (P_NEG / (2.0 * NEG))
               + reg_acc * (REG_SCALE / 2.0)) * (1.0 / B)
    part_v[...] = partial
    pltpu.sync_copy(part_v, out_hbm.at[wid])


_sc_loss = pl.kernel(
    _sc_body,
    out_type=jax.ShapeDtypeStruct((NW, L), jnp.float32),
    mesh=plsc.VectorSubcoreMesh(core_axis_name="c", subcore_axis_name="s",
                                num_cores=NC, num_subcores=NS),
    scratch_types=[
        pltpu.VMEM((CH,), jnp.int32),
        pltpu.VMEM((CH,), jnp.int32),
        pltpu.VMEM((CH,), jnp.int32),
        pltpu.VMEM((CH,), jnp.float32),
        pltpu.VMEM((CH, DIM), jnp.float32),
        pltpu.VMEM((CH, DIM), jnp.float32),
        pltpu.VMEM((CH, DIM), jnp.float32),
        pltpu.VMEM((CH, DIM), jnp.float32),
        pltpu.VMEM((CH, DIM), jnp.float32),
        pltpu.VMEM((CH, DIM), jnp.float32),
        pltpu.VMEM((CH,), jnp.float32),
        pltpu.VMEM((2, L), jnp.float32),
        pltpu.VMEM((L,), jnp.float32),
        pltpu.SemaphoreType.DMA,
    ],
)


def kernel(h, r, t, w, n_hn, n_rel_hn, n_t, n_h, n_rel_tn, n_tn,
           s_h, s_r, s_t, s_w, ent_real, ent_img, rel_real, rel_img,
           lin_w, lin_b):
    i32 = jnp.int32
    lwb = jnp.concatenate(
        [jnp.full((1, L), lin_w[0, 0], jnp.float32),
         jnp.full((1, L), lin_b[0], jnp.float32)], axis=0)
    out = _sc_loss(
        h.astype(i32), r.astype(i32), t.astype(i32), w,
        n_hn.reshape(-1).astype(i32), n_rel_hn.reshape(-1).astype(i32),
        n_t.reshape(-1).astype(i32),
        n_h.reshape(-1).astype(i32), n_rel_tn.reshape(-1).astype(i32),
        n_tn.reshape(-1).astype(i32),
        ent_real, ent_img, rel_real, rel_img, lwb)
    return jnp.sum(out)


# speedup vs baseline: 3.0159x; 3.0159x over previous
import functools

import jax
import jax.numpy as jnp
from jax import lax
from jax.experimental import pallas as pl
from jax.experimental.pallas import tpu as pltpu
from jax.experimental.pallas import tpu_sc as plsc

NUM_CONS = 100000
NUM_RELS = 1000
DIM = 128
B = 4096
NEG = 10
REG_SCALE = 0.0005
P_NEG = 1.0

NC = 2
NS = 16
L = 16
NW = NC * NS
CH = 128
POS_PER_W = B // NW
NEG_PER_W = (B * NEG) // NW
NEG_CHUNKS = NEG_PER_W // CH
DG = DIM // L


def _sc_body(h_hbm, r_hbm, t_hbm, w_hbm,
             hn_e_hbm, hn_r_hbm, hn_t_hbm,
             tn_e_hbm, tn_r_hbm, tn_t_hbm,
             ent_re_hbm, ent_im_hbm, rel_re_hbm, rel_im_hbm,
             lwb_hbm,
             out_hbm,
             ia_v, ib_v, ic_v, w_v,
             hre_v, him_v, rre_v, rim_v, tre_v, tim_v,
             lwb_v, part_v, sem):
    cid = lax.axis_index("c")
    sid = lax.axis_index("s")
    wid = sid * NC + cid

    pltpu.sync_copy(lwb_hbm, lwb_v)
    lw = lwb_v[0, :]
    lb = lwb_v[1, :]
    lanes = jnp.arange(L, dtype=jnp.int32)

    def gather_chunk(eidx_hbm, ridx_hbm, tidx_hbm, base):
        base = pl.multiple_of(base, 8)
        pltpu.sync_copy(eidx_hbm.at[pl.ds(base, CH)], ia_v)
        pltpu.sync_copy(ridx_hbm.at[pl.ds(base, CH)], ib_v)
        pltpu.sync_copy(tidx_hbm.at[pl.ds(base, CH)], ic_v)
        cps = [
            pltpu.make_async_copy(ent_re_hbm.at[ia_v], hre_v, sem),
            pltpu.make_async_copy(ent_im_hbm.at[ia_v], him_v, sem),
            pltpu.make_async_copy(rel_re_hbm.at[ib_v], rre_v, sem),
            pltpu.make_async_copy(rel_im_hbm.at[ib_v], rim_v, sem),
            pltpu.make_async_copy(ent_re_hbm.at[ic_v], tre_v, sem),
            pltpu.make_async_copy(ent_im_hbm.at[ic_v], tim_v, sem),
        ]
        for cp in cps:
            cp.start()
        for cp in cps:
            cp.wait()

    zero = jnp.zeros((L,), jnp.float32)

    def chunk_compute(positive, acc0, reg0):
        acc = acc0
        reg = reg0
        for g in range(CH // L):
            rows = g * L + lanes

            def dstep(d, carry):
                sc, rg = carry
                dvec = (d + lanes) & (DIM - 1)
                idx = [rows, dvec]
                a = plsc.load_gather(hre_v, idx)
                bb = plsc.load_gather(him_v, idx)
                c = plsc.load_gather(rre_v, idx)
                e = plsc.load_gather(rim_v, idx)
                f = plsc.load_gather(tre_v, idx)
                gg = plsc.load_gather(tim_v, idx)
                sc = sc + c * (a * f + bb * gg) + e * (a * gg - bb * f)
                if positive:
                    rg = rg + (a * a + bb * bb + c * c
                               + e * e + f * f + gg * gg)
                return (sc, rg)

            sc, reg = lax.fori_loop(0, DIM, dstep, (zero, reg), unroll=2)
            z = sc * lw + lb
            p = 1.0 / (1.0 + jnp.exp(-z))
            if positive:
                dd = p - w_v[pl.ds(g * L, L)]
            else:
                dd = p
            acc = acc + dd * dd
        return acc, reg

    pos_base = pl.multiple_of(wid * POS_PER_W, 8)
    pltpu.sync_copy(w_hbm.at[pl.ds(pos_base, CH)], w_v)
    gather_chunk(h_hbm, r_hbm, t_hbm, pos_base)
    pos_acc, reg_acc = chunk_compute(True, zero, zero)

    def neg_loop(eidx, ridx, tidx):
        def chunk(c, acc):
            gather_chunk(eidx, ridx, tidx, wid * NEG_PER_W + c * CH)
            acc, _ = chunk_compute(False, acc, zero)
            return acc
        return lax.fori_loop(0, NEG_CHUNKS, chunk, zero)

    hn_acc = neg_loop(hn_e_hbm, hn_r_hbm, hn_t_hbm)
    tn_acc = neg_loop(tn_e_hbm, tn_r_hbm, tn_t_hbm)

    partial = (pos_acc
               + (hn_acc + tn_acc) * (P_NEG / (2.0 * NEG))
               + reg_acc * (REG_SCALE / 2.0)) * (1.0 / B)
    part_v[...] = partial
    pltpu.sync_copy(part_v, out_hbm.at[wid])


_sc_loss = pl.kernel(
    _sc_body,
    out_type=jax.ShapeDtypeStruct((NW, L), jnp.float32),
    mesh=plsc.VectorSubcoreMesh(core_axis_name="c", subcore_axis_name="s",
                                num_cores=NC, num_subcores=NS),
    scratch_types=[
        pltpu.VMEM((CH,), jnp.int32),
        pltpu.VMEM((CH,), jnp.int32),
        pltpu.VMEM((CH,), jnp.int32),
        pltpu.VMEM((CH,), jnp.float32),
        pltpu.VMEM((CH, DIM), jnp.float32),
        pltpu.VMEM((CH, DIM), jnp.float32),
        pltpu.VMEM((CH, DIM), jnp.float32),
        pltpu.VMEM((CH, DIM), jnp.float32),
        pltpu.VMEM((CH, DIM), jnp.float32),
        pltpu.VMEM((CH, DIM), jnp.float32),
        pltpu.VMEM((2, L), jnp.float32),
        pltpu.VMEM((L,), jnp.float32),
        pltpu.SemaphoreType.DMA,
    ],
    compiler_params=pltpu.CompilerParams(needs_layout_passes=False),
)


def kernel(h, r, t, w, n_hn, n_rel_hn, n_t, n_h, n_rel_tn, n_tn,
           s_h, s_r, s_t, s_w, ent_real, ent_img, rel_real, rel_img,
           lin_w, lin_b):
    i32 = jnp.int32
    lwb = jnp.concatenate(
        [jnp.full((1, L), lin_w[0, 0], jnp.float32),
         jnp.full((1, L), lin_b[0], jnp.float32)], axis=0)
    out = _sc_loss(
        h.astype(i32), r.astype(i32), t.astype(i32), w,
        n_hn.reshape(-1).astype(i32), n_rel_hn.reshape(-1).astype(i32),
        n_t.reshape(-1).astype(i32),
        n_h.reshape(-1).astype(i32), n_rel_tn.reshape(-1).astype(i32),
        n_tn.reshape(-1).astype(i32),
        ent_real, ent_img, rel_real, rel_img, lwb)
    return jnp.sum(out)
